# baseline (speedup 1.0000x reference)
import jax
import jax.numpy as jnp
import numpy as np
from jax.experimental import pallas as pl
from jax.experimental.pallas import tpu as pltpu

B, S, H = 16, 38, 768
IMG = 512
D = H + IMG * 2
NH = 4
DK = D // NH
NP = 64
NQ = B * S

_HI = jax.lax.Precision.HIGHEST


def _dot(a, b, dims=((1,), (0,))):
    return jax.lax.dot_general(a, b, (dims, ((), ())), precision=_HI,
                               preferred_element_type=jnp.float32)


def _prep_kernel(proto_ref, wq_ref, bq_ref, wk_ref, bk_ref, wv_ref, bv_ref,
                 t_ref, sb_ref, v0_ref):
    p = proto_ref[...]
    k0 = _dot(p, wk_ref[...]) + bk_ref[...]
    v0_ref[...] = _dot(p, wv_ref[...]) + bv_ref[...]
    for h in range(NH):
        hs = slice(h * DK, (h + 1) * DK)
        k0h = k0[:, hs]
        t_ref[:, h * NP:(h + 1) * NP] = _dot(wq_ref[:, hs], k0h,
                                             ((1,), (1,)))
        sb_ref[:, h * NP:(h + 1) * NP] = _dot(bq_ref[:, hs], k0h,
                                              ((1,), (1,)))


def _prep_w0_kernel(v0_ref, wo0_ref, wo1_ref, wo2_ref, wo3_ref, wfcb_ref,
                    bo_ref, bfc_ref, w0_ref, vb_ref):
    wfcb = wfcb_ref[...]
    wo_refs = (wo0_ref, wo1_ref, wo2_ref, wo3_ref)
    for h in range(NH):
        hs = slice(h * DK, (h + 1) * DK)
        u0h = _dot(v0_ref[:, hs], wo_refs[h][...])
        w0_ref[h * NP:(h + 1) * NP, :] = _dot(u0h, wfcb)
    vb_ref[...] = _dot(bo_ref[...], wfcb) + bfc_ref[...]


def _main_kernel(cf_ref, t_ref, sb_ref, w0_ref, wfct_ref, vb_ref,
                 wqa_ref, bqa_ref, spos_ref, epos_ref,
                 slog_ref, elog_ref, loss_ref):
    cf = cf_ref[...]
    scores = _dot(cf, t_ref[...]) + sb_ref[...]
    iota = jax.lax.broadcasted_iota(jnp.int32, (NQ, NP), 1)
    wsel = jnp.zeros((NQ, H), dtype=jnp.float32)
    for h in range(NH):
        sh = scores[:, h * NP:(h + 1) * NP]
        m = jnp.max(sh, axis=1, keepdims=True)
        idx = jnp.min(jnp.where(sh == m, iota, NP), axis=1, keepdims=True)
        onehot = (iota == idx).astype(jnp.float32)
        wsel = wsel + _dot(onehot, w0_ref[h * NP:(h + 1) * NP, :])
    reduced = _dot(cf, wfct_ref[...]) + wsel + vb_ref[...]
    logits = _dot(reduced, wqa_ref[...]) + bqa_ref[...]
    slog_ref[...] = logits[:, 0:1]
    elog_ref[...] = logits[:, 1:2]

    rowi = jax.lax.broadcasted_iota(jnp.int32, (B, NQ), 1)
    bi = jax.lax.broadcasted_iota(jnp.int32, (B, NQ), 0)
    onb = ((rowi >= bi * S) & (rowi < (bi + 1) * S)).astype(jnp.float32)
    sums = _dot(onb, jnp.exp(logits))
    lse = jnp.log(sums)
    ons = (rowi == bi * S + spos_ref[...]).astype(jnp.float32)
    one = (rowi == bi * S + epos_ref[...]).astype(jnp.float32)
    sel_s = _dot(ons, logits[:, 0:1])
    sel_e = _dot(one, logits[:, 1:2])
    loss = 0.5 * (jnp.mean(lse[:, 0:1] - sel_s) +
                  jnp.mean(lse[:, 1:2] - sel_e))
    loss_ref[...] = jnp.reshape(loss, (1, 1))


def _f32(shape):
    return jax.ShapeDtypeStruct(shape, jnp.float32)


def kernel(combined_features, attention_mask, start_positions, end_positions,
           prototype_vectors, Wq, bq, Wk, bk, Wv, bv, Wo, bo, Wfc, bfc,
           Wqa, bqa):
    cf2d = combined_features.reshape(NQ, D)
    row = lambda x: x.reshape(1, -1)

    t, sbias, v0 = pl.pallas_call(
        _prep_kernel,
        out_shape=(_f32((D, NH * NP)), _f32((1, NH * NP)), _f32((NP, D))),
    )(prototype_vectors, Wq, row(bq), Wk, row(bk), Wv, row(bv))

    spos = start_positions.astype(jnp.int32).reshape(B, 1)
    epos = end_positions.astype(jnp.int32).reshape(B, 1)

    w0, vbias = pl.pallas_call(
        _prep_w0_kernel,
        out_shape=(_f32((NH * NP, H)), _f32((1, H))),
    )(v0,
      Wo[0 * DK:1 * DK, :], Wo[1 * DK:2 * DK, :],
      Wo[2 * DK:3 * DK, :], Wo[3 * DK:4 * DK, :],
      Wfc[D:, :], row(bo), row(bfc))

    slog, elog, loss = pl.pallas_call(
        _main_kernel,
        out_shape=(_f32((NQ, 1)), _f32((NQ, 1)), _f32((1, 1))),
    )(cf2d, t, sbias, w0, Wfc[:D, :], vbias, Wqa, row(bqa),
      spos, epos)

    return loss[0, 0], slog.reshape(B, S), elog.reshape(B, S)
